# Initial kernel scaffold; baseline (speedup 1.0000x reference)
import jax
import jax.numpy as jnp
from jax.experimental import pallas as pl


def kernel(x_nchw, conv_w, gamma, beta):
    raise NotImplementedError("write your pallas kernel here")


# speedup vs baseline: 1.1656x; 1.1656x over previous
import functools

import jax
import jax.numpy as jnp
from jax.experimental import pallas as pl
from jax.experimental.pallas import tpu as pltpu

_BN_EPS = 1e-5
_NEG_SLOPE = 0.01


def _fused_body(x_ref, w_ref, g_ref, b_ref, o_ref,
                y_scr, sum_scr, ssq_scr, a_scr, bias_scr, *, n_batches, m):
    p = pl.program_id(0)
    n = pl.program_id(1)

    @pl.when(p == 0)
    def _stats_phase():
        @pl.when(n == 0)
        def _():
            sum_scr[...] = jnp.zeros_like(sum_scr)
            ssq_scr[...] = jnp.zeros_like(ssq_scr)

        y = jnp.dot(w_ref[...], x_ref[0], preferred_element_type=jnp.float32)
        y_scr[n] = y
        sum_scr[...] += jnp.sum(y, axis=1, keepdims=True)
        ssq_scr[...] += jnp.sum(y * y, axis=1, keepdims=True)

    @pl.when(p == 1)
    def _apply_phase():
        @pl.when(n == 0)
        def _():
            inv_m = 1.0 / m
            mean = sum_scr[...] * inv_m
            var = jnp.maximum(ssq_scr[...] * inv_m - mean * mean, 0.0)
            a = g_ref[...] * jax.lax.rsqrt(var + _BN_EPS)
            a_scr[...] = a
            bias_scr[...] = b_ref[...] - a * mean

        z = y_scr[n] * a_scr[...] + bias_scr[...]
        o_ref[0] = jnp.maximum(z, _NEG_SLOPE * z).astype(o_ref.dtype)


@jax.jit
def _conv_bn_leaky(x_nchw, conv_w, gamma, beta):
    N, Cin, H, W = x_nchw.shape
    Cout = conv_w.shape[0]
    HW = H * W
    m = float(N * HW)

    x3 = x_nchw.reshape(N, Cin, HW)
    w2 = conv_w.reshape(Cout, Cin)
    g1 = gamma.astype(jnp.float32).reshape(Cout, 1)
    b1 = beta.astype(jnp.float32).reshape(Cout, 1)

    body = functools.partial(_fused_body, n_batches=N, m=m)
    out3 = pl.pallas_call(
        body,
        out_shape=jax.ShapeDtypeStruct((N, Cout, HW), x3.dtype),
        grid=(2, N),
        in_specs=[
            pl.BlockSpec((1, Cin, HW),
                         lambda p, n: (n * (1 - p) + (N - 1) * p, 0, 0)),
            pl.BlockSpec((Cout, Cin), lambda p, n: (0, 0)),
            pl.BlockSpec((Cout, 1), lambda p, n: (0, 0)),
            pl.BlockSpec((Cout, 1), lambda p, n: (0, 0)),
        ],
        out_specs=pl.BlockSpec((1, Cout, HW), lambda p, n: (n * p, 0, 0)),
        scratch_shapes=[
            pltpu.VMEM((N, Cout, HW), jnp.float32),
            pltpu.VMEM((Cout, 1), jnp.float32),
            pltpu.VMEM((Cout, 1), jnp.float32),
            pltpu.VMEM((Cout, 1), jnp.float32),
            pltpu.VMEM((Cout, 1), jnp.float32),
        ],
        compiler_params=pltpu.CompilerParams(
            dimension_semantics=("arbitrary", "arbitrary"),
            vmem_limit_bytes=60 * 1024 * 1024),
    )(x3, w2, g1, b1)

    return out3.reshape(N, Cout, H, W)


def kernel(x_nchw, conv_w, gamma, beta):
    return [_conv_bn_leaky(x_nchw, conv_w, gamma, beta)]
